# baseline (speedup 1.0000x reference)
import functools

import jax
import jax.numpy as jnp
from jax import lax
from jax.experimental import pallas as pl
from jax.experimental.pallas import tpu as pltpu
from jax.experimental.pallas import tpu_sc as plsc

VOCAB = 1000
EMBED_DIM = 4000
BATCH = 4096

_info = plsc.get_sparse_core_info()
_NC, _NS = _info.num_cores, _info.num_subcores
NW = _NC * _NS
BPW = BATCH // NW
CH = 8
NCHUNK = BPW // CH


def _body(idx_hbm, table_hbm, out_hbm, idx_v, buf0, buf1,
          gsem0, gsem1, osem0, osem1):
    wid = lax.axis_index("s") * _NC + lax.axis_index("c")
    base = wid * BPW

    pltpu.sync_copy(idx_hbm.at[wid], idx_v)

    bufs = (buf0, buf1)
    gsems = (gsem0, gsem1)
    osems = (osem0, osem1)

    def gather(c, s):
        return pltpu.async_copy(table_hbm.at[idx_v.at[c]], bufs[s], gsems[s])

    gc = [gather(0, 0), gather(1, 1)]
    oc = [None, None]
    for c in range(NCHUNK):
        s = c % 2
        gc[s].wait()
        oc[s] = pltpu.async_copy(
            bufs[s], out_hbm.at[pl.ds(base + c * CH, CH)], osems[s])
        nxt = c + 2
        if nxt < NCHUNK:
            oc[s].wait()
            gc[s] = gather(nxt, s)
    oc[0].wait()
    oc[1].wait()


@functools.partial(jax.jit, static_argnums=())
def _gather_call(idx, table):
    mesh = plsc.VectorSubcoreMesh(core_axis_name="c", subcore_axis_name="s")
    fn = functools.partial(
        pl.kernel,
        mesh=mesh,
        compiler_params=pltpu.CompilerParams(use_tc_tiling_on_sc=False),
        out_type=jax.ShapeDtypeStruct((BATCH, EMBED_DIM), jnp.float32),
        scratch_types=[
            pltpu.VMEM((NCHUNK, CH), jnp.int32),
            pltpu.VMEM((CH, EMBED_DIM), jnp.float32),
            pltpu.VMEM((CH, EMBED_DIM), jnp.float32),
            pltpu.SemaphoreType.DMA,
            pltpu.SemaphoreType.DMA,
            pltpu.SemaphoreType.DMA,
            pltpu.SemaphoreType.DMA,
        ],
    )(_body)
    return fn(idx, table)


def kernel(tokens, embedding_table):
    idx = tokens.astype(jnp.int32).reshape(NW, NCHUNK, CH)
    return _gather_call(idx, embedding_table)


# speedup vs baseline: 1.0044x; 1.0044x over previous
import functools

import jax
import jax.numpy as jnp
from jax import lax
from jax.experimental import pallas as pl
from jax.experimental.pallas import tpu as pltpu
from jax.experimental.pallas import tpu_sc as plsc

VOCAB = 1000
EMBED_DIM = 4000
BATCH = 4096

_info = plsc.get_sparse_core_info()
_NC, _NS = _info.num_cores, _info.num_subcores
NW = _NC * _NS
BPW = BATCH // NW
CH = 8
NCHUNK = BPW // CH


NBUF = 4


def _body(idx_hbm, table_hbm, out_hbm, idx_v, *bufs_and_sems):
    bufs = bufs_and_sems[:NBUF]
    gsems = bufs_and_sems[NBUF:2 * NBUF]
    osems = bufs_and_sems[2 * NBUF:3 * NBUF]

    wid = lax.axis_index("s") * _NC + lax.axis_index("c")
    base = wid * BPW

    pltpu.sync_copy(idx_hbm.at[wid], idx_v)

    def gather(c, s):
        return pltpu.async_copy(table_hbm.at[idx_v.at[c]], bufs[s], gsems[s])

    gc = [gather(s, s) for s in range(NBUF)]
    oc = [None] * NBUF
    for c in range(NCHUNK):
        s = c % NBUF
        gc[s].wait()
        oc[s] = pltpu.async_copy(
            bufs[s], out_hbm.at[pl.ds(base + c * CH, CH)], osems[s])
        nxt = c + NBUF
        if nxt < NCHUNK:
            oc[s].wait()
            gc[s] = gather(nxt, s)
    for s in range(NBUF):
        oc[s].wait()


@functools.partial(jax.jit, static_argnums=())
def _gather_call(idx, table):
    mesh = plsc.VectorSubcoreMesh(core_axis_name="c", subcore_axis_name="s")
    fn = functools.partial(
        pl.kernel,
        mesh=mesh,
        compiler_params=pltpu.CompilerParams(use_tc_tiling_on_sc=False),
        out_type=jax.ShapeDtypeStruct((BATCH, EMBED_DIM), jnp.float32),
        scratch_types=(
            [pltpu.VMEM((NCHUNK, CH), jnp.int32)]
            + [pltpu.VMEM((CH, EMBED_DIM), jnp.float32)] * NBUF
            + [pltpu.SemaphoreType.DMA] * (2 * NBUF)
        ),
    )(_body)
    return fn(idx, table)


def kernel(tokens, embedding_table):
    idx = tokens.astype(jnp.int32).reshape(NW, NCHUNK, CH)
    return _gather_call(idx, embedding_table)


# speedup vs baseline: 1.5489x; 1.5422x over previous
import functools

import jax
import jax.numpy as jnp
from jax import lax
from jax.experimental import pallas as pl
from jax.experimental.pallas import tpu as pltpu
from jax.experimental.pallas import tpu_sc as plsc

VOCAB = 1000
EMBED_DIM = 4000
PAD_DIM = 4096
BATCH = 4096

_info = plsc.get_sparse_core_info()
_NC, _NS = _info.num_cores, _info.num_subcores
NW = _NC * _NS
BPW = BATCH // NW
CH = 8
NCHUNK = BPW // CH
NBUF = 3


def _body(idx_hbm, table_hbm, out_hbm, idx_v, *bufs_and_sems):
    bufs = bufs_and_sems[:NBUF]
    gsems = bufs_and_sems[NBUF:2 * NBUF]
    osems = bufs_and_sems[2 * NBUF:3 * NBUF]

    wid = lax.axis_index("s") * _NC + lax.axis_index("c")
    base = wid * BPW

    pltpu.sync_copy(idx_hbm.at[pl.ds(base, BPW)], idx_v)

    def gather(c, s):
        return pltpu.async_copy(
            table_hbm.at[idx_v.at[pl.ds(c * CH, CH)]], bufs[s], gsems[s])

    gc = [gather(s, s) for s in range(NBUF)]
    oc = [None] * NBUF
    for c in range(NCHUNK):
        s = c % NBUF
        gc[s].wait()
        oc[s] = pltpu.async_copy(
            bufs[s], out_hbm.at[pl.ds(base + c * CH, CH)], osems[s])
        nxt = c + NBUF
        if nxt < NCHUNK:
            oc[s].wait()
            gc[s] = gather(nxt, s)
    for s in range(NBUF):
        oc[s].wait()


def _make_call():
    mesh = plsc.VectorSubcoreMesh(core_axis_name="c", subcore_axis_name="s")
    return functools.partial(
        pl.kernel,
        mesh=mesh,
        out_type=jax.ShapeDtypeStruct((BATCH, PAD_DIM), jnp.float32),
        scratch_types=(
            [pltpu.VMEM((BPW,), jnp.int32)]
            + [pltpu.VMEM((CH, PAD_DIM), jnp.float32)] * NBUF
            + [pltpu.SemaphoreType.DMA] * (2 * NBUF)
        ),
    )(_body)


_gather_call = _make_call()


def kernel(tokens, embedding_table):
    idx = tokens.astype(jnp.int32)
    table_p = jnp.pad(embedding_table, ((0, 0), (0, PAD_DIM - EMBED_DIM)))
    out_p = _gather_call(idx, table_p)
    return out_p[:, :EMBED_DIM]
